# baseline (speedup 1.0000x reference)
import jax
import jax.numpy as jnp
from jax.experimental import pallas as pl

_TOKENS = 8192
_DIM = 2048
_EXPERTS = 64
_BT = 512


def _gate_kernel(x_ref, w_ref, out_ref):
    logits = jax.lax.dot_general(
        x_ref[...], w_ref[...],
        dimension_numbers=(((1,), (1,)), ((), ())),
        preferred_element_type=jnp.float32)
    m = jnp.max(logits, axis=1, keepdims=True)
    e = jnp.exp(logits - m)
    out_ref[...] = e / jnp.sum(e, axis=1, keepdims=True)


def kernel(inputs, wg):
    return pl.pallas_call(
        _gate_kernel,
        grid=(_TOKENS // _BT,),
        in_specs=[
            pl.BlockSpec((_BT, _DIM), lambda i: (i, 0)),
            pl.BlockSpec((_EXPERTS, _DIM), lambda i: (0, 0)),
        ],
        out_specs=pl.BlockSpec((_BT, _EXPERTS), lambda i: (i, 0)),
        out_shape=jax.ShapeDtypeStruct((_TOKENS, _EXPERTS), jnp.float32),
    )(inputs, wg)


# speedup vs baseline: 1.0947x; 1.0947x over previous
import jax
import jax.numpy as jnp
from jax.experimental import pallas as pl

_TOKENS = 8192
_DIM = 2048
_EXPERTS = 64
_BT = 2048


def _gate_kernel(x_ref, w_ref, out_ref):
    logits = jax.lax.dot_general(
        x_ref[...], w_ref[...],
        dimension_numbers=(((1,), (1,)), ((), ())),
        preferred_element_type=jnp.float32)
    m = jnp.max(logits, axis=1, keepdims=True)
    e = jnp.exp(logits - m)
    out_ref[...] = e / jnp.sum(e, axis=1, keepdims=True)


def kernel(inputs, wg):
    return pl.pallas_call(
        _gate_kernel,
        grid=(_TOKENS // _BT,),
        in_specs=[
            pl.BlockSpec((_BT, _DIM), lambda i: (i, 0)),
            pl.BlockSpec((_EXPERTS, _DIM), lambda i: (0, 0)),
        ],
        out_specs=pl.BlockSpec((_BT, _EXPERTS), lambda i: (i, 0)),
        out_shape=jax.ShapeDtypeStruct((_TOKENS, _EXPERTS), jnp.float32),
    )(inputs, wg)


# speedup vs baseline: 1.0962x; 1.0014x over previous
import jax
import jax.numpy as jnp
from jax.experimental import pallas as pl

_TOKENS = 8192
_DIM = 2048
_EXPERTS = 64
_BT = 1024


def _softmax_rows(logits):
    m = jnp.max(logits, axis=1, keepdims=True)
    e = jnp.exp(logits - m)
    return e / jnp.sum(e, axis=1, keepdims=True)


def _gate_kernel(x0_ref, x1_ref, w_ref, out_ref):
    w = w_ref[...]
    dn = (((1,), (1,)), ((), ()))
    l0 = jax.lax.dot_general(x0_ref[...], w, dimension_numbers=dn,
                             preferred_element_type=jnp.float32)
    out_ref[0:_BT, :] = _softmax_rows(l0)
    l1 = jax.lax.dot_general(x1_ref[...], w, dimension_numbers=dn,
                             preferred_element_type=jnp.float32)
    out_ref[_BT:2 * _BT, :] = _softmax_rows(l1)


def kernel(inputs, wg):
    return pl.pallas_call(
        _gate_kernel,
        grid=(_TOKENS // (2 * _BT),),
        in_specs=[
            pl.BlockSpec((_BT, _DIM), lambda i: (2 * i, 0)),
            pl.BlockSpec((_BT, _DIM), lambda i: (2 * i + 1, 0)),
            pl.BlockSpec((_EXPERTS, _DIM), lambda i: (0, 0)),
        ],
        out_specs=pl.BlockSpec((2 * _BT, _EXPERTS), lambda i: (i, 0)),
        out_shape=jax.ShapeDtypeStruct((_TOKENS, _EXPERTS), jnp.float32),
    )(inputs, inputs, wg)


# speedup vs baseline: 1.1737x; 1.0707x over previous
import jax
import jax.numpy as jnp
from jax.experimental import pallas as pl
from jax.experimental.pallas import tpu as pltpu

_TOKENS = 8192
_DIM = 2048
_EXPERTS = 64
_BT = 1024


def _gate_kernel(x_ref, w_ref, out_ref):
    logits = jax.lax.dot_general(
        x_ref[...], w_ref[...],
        dimension_numbers=(((1,), (1,)), ((), ())),
        preferred_element_type=jnp.float32)
    m = jnp.max(logits, axis=1, keepdims=True)
    e = jnp.exp(logits - m)
    out_ref[...] = e / jnp.sum(e, axis=1, keepdims=True)


def kernel(inputs, wg):
    return pl.pallas_call(
        _gate_kernel,
        grid=(_TOKENS // _BT,),
        in_specs=[
            pl.BlockSpec((_BT, _DIM), lambda i: (i, 0)),
            pl.BlockSpec((_EXPERTS, _DIM), lambda i: (0, 0)),
        ],
        out_specs=pl.BlockSpec((_BT, _EXPERTS), lambda i: (i, 0)),
        out_shape=jax.ShapeDtypeStruct((_TOKENS, _EXPERTS), jnp.float32),
        compiler_params=pltpu.CompilerParams(
            dimension_semantics=("parallel",)),
    )(inputs, wg)
